# baseline (speedup 1.0000x reference)
import jax
import jax.numpy as jnp
from jax.experimental import pallas as pl

_CHUNK = 1000


def _gcn_body(a_ref, hs_ref, ws2r_ref, wr2s_ref, out_ref):
    R, S = a_ref.shape
    d = hs_ref.shape[1]
    L = ws2r_ref.shape[0]
    n_chunks = R // _CHUNK

    hs_bf = hs_ref[...].astype(jnp.bfloat16)
    deg_s_row = jnp.zeros((1, S), jnp.float32)

    for l in range(L):
        w1 = ws2r_ref[l].astype(jnp.bfloat16)
        last = l == L - 1

        def chunk_step(i, carry):
            tT_acc, ds_acc = carry
            blk = a_ref[pl.ds(i * _CHUNK, _CHUNK), :]
            m = jax.lax.dot_general(
                blk, hs_bf, (((1,), (0,)), ((), ())),
                preferred_element_type=jnp.float32)
            deg_blk = jnp.sum(blk, axis=1, keepdims=True,
                              dtype=jnp.float32) + 1e-8
            hr_blk = jnp.maximum(
                jax.lax.dot_general(
                    m.astype(jnp.bfloat16), w1, (((1,), (0,)), ((), ())),
                    preferred_element_type=jnp.float32),
                0.0) / deg_blk
            if last:
                out_ref[pl.ds(i * _CHUNK, _CHUNK), :] = hr_blk
            else:
                tT_acc = tT_acc + jax.lax.dot_general(
                    hr_blk.astype(jnp.bfloat16), blk, (((0,), (0,)), ((), ())),
                    preferred_element_type=jnp.float32)
            if l == 0:
                ds_acc = ds_acc + jnp.sum(blk, axis=0, keepdims=True,
                                          dtype=jnp.float32)
            return tT_acc, ds_acc

        tT, ds = jax.lax.fori_loop(
            0, n_chunks, chunk_step,
            (jnp.zeros((d, S), jnp.float32), deg_s_row))
        if l == 0:
            deg_s_row = ds + 1e-8
        if last:
            break
        w2 = wr2s_ref[l].astype(jnp.bfloat16)
        hsT = jnp.maximum(
            jax.lax.dot_general(
                w2, (tT / deg_s_row).astype(jnp.bfloat16),
                (((0,), (0,)), ((), ())),
                preferred_element_type=jnp.float32),
            0.0)
        hs_bf = hsT.T.astype(jnp.bfloat16)


@jax.jit
def kernel(h_s, A_rs, r_embed, W_s2r, W_r2s):
    del r_embed
    R = A_rs.shape[0]
    d = h_s.shape[1]
    a16 = A_rs.astype(jnp.bfloat16)
    return pl.pallas_call(
        _gcn_body,
        out_shape=jax.ShapeDtypeStruct((R, d), jnp.float32),
    )(a16, h_s, W_s2r, W_r2s)


# speedup vs baseline: 1.0883x; 1.0883x over previous
import jax
import jax.numpy as jnp
from jax.experimental import pallas as pl
from jax.experimental.pallas import tpu as pltpu

_C = 1000
_PAD = 8


def _gcn_body(a_ref, hs_ref, w1_ref, w2_ref, out_ref,
              hsa_ref, tT_ref, degs_ref):
    l = pl.program_id(0)
    i = pl.program_id(1)
    nc = pl.num_programs(1)
    S, d = hs_ref.shape

    @pl.when((l == 0) & (i == 0))
    def _init_hs():
        hsa_ref[:, :d] = hs_ref[...].astype(jnp.bfloat16)
        hsa_ref[:, d:] = jnp.ones((S, _PAD), jnp.bfloat16)

    @pl.when(i == 0)
    def _reset_acc():
        tT_ref[...] = jnp.zeros_like(tT_ref)

    blk = a_ref[...]
    m_aug = jax.lax.dot_general(
        blk, hsa_ref[...], (((1,), (0,)), ((), ())),
        preferred_element_type=jnp.float32)
    deg_r = m_aug[:, d:d + 1] + 1e-8
    w1 = w1_ref[0].astype(jnp.bfloat16)
    hr = jnp.maximum(
        jax.lax.dot_general(
            m_aug[:, :d].astype(jnp.bfloat16), w1, (((1,), (0,)), ((), ())),
            preferred_element_type=jnp.float32),
        0.0) / deg_r
    out_ref[...] = hr

    @pl.when(l < 2)
    def _accumulate():
        hr_aug = jnp.concatenate(
            [hr.astype(jnp.bfloat16), jnp.ones((_C, _PAD), jnp.bfloat16)],
            axis=1)
        tT_ref[...] += jax.lax.dot_general(
            hr_aug, blk, (((0,), (0,)), ((), ())),
            preferred_element_type=jnp.float32)

    @pl.when((l < 2) & (i == nc - 1))
    def _finish_layer():
        @pl.when(l == 0)
        def _save_deg_s():
            degs_ref[...] = tT_ref[d:d + 1, :] + 1e-8
        tT = tT_ref[:d, :] / degs_ref[...]
        w2 = w2_ref[0].astype(jnp.bfloat16)
        hsT = jnp.maximum(
            jax.lax.dot_general(
                w2, tT.astype(jnp.bfloat16), (((0,), (0,)), ((), ())),
                preferred_element_type=jnp.float32),
            0.0)
        hsa_ref[:, :d] = hsT.T.astype(jnp.bfloat16)


@jax.jit
def kernel(h_s, A_rs, r_embed, W_s2r, W_r2s):
    del r_embed
    R, S = A_rs.shape
    d = h_s.shape[1]
    L = W_s2r.shape[0]
    a16 = A_rs.astype(jnp.bfloat16)
    return pl.pallas_call(
        _gcn_body,
        grid=(L, R // _C),
        in_specs=[
            pl.BlockSpec((_C, S), lambda l, i: (i, 0)),
            pl.BlockSpec((S, d), lambda l, i: (0, 0)),
            pl.BlockSpec((1, d, d), lambda l, i: (l, 0, 0)),
            pl.BlockSpec((1, d, d), lambda l, i: (l, 0, 0)),
        ],
        out_specs=pl.BlockSpec((_C, d), lambda l, i: (i, 0)),
        out_shape=jax.ShapeDtypeStruct((R, d), jnp.float32),
        scratch_shapes=[
            pltpu.VMEM((S, d + _PAD), jnp.bfloat16),
            pltpu.VMEM((d + _PAD, S), jnp.float32),
            pltpu.VMEM((1, S), jnp.float32),
        ],
    )(a16, h_s, W_s2r, W_r2s)


# speedup vs baseline: 1.2143x; 1.1158x over previous
import jax
import jax.numpy as jnp
from jax.experimental import pallas as pl
from jax.experimental.pallas import tpu as pltpu

_C = 1000
_PAD = 8


def _gcn_body(a_ref, hs_ref, w1_ref, w2_ref, out_ref,
              a16_ref, hsa_ref, tT_ref, degs_ref):
    l = pl.program_id(0)
    i = pl.program_id(1)
    nc = pl.num_programs(1)
    S, d = hs_ref.shape

    @pl.when((l == 0) & (i == 0))
    def _init_hs():
        hsa_ref[:, :d] = hs_ref[...].astype(jnp.bfloat16)
        hsa_ref[:, d:] = jnp.ones((S, _PAD), jnp.bfloat16)

    @pl.when(i == 0)
    def _reset_acc():
        tT_ref[...] = jnp.zeros_like(tT_ref)

    @pl.when(l == 0)
    def _stash_block():
        a16_ref[pl.ds(i * _C, _C), :] = a_ref[...].astype(jnp.bfloat16)

    blk = a16_ref[pl.ds(i * _C, _C), :]
    m_aug = jax.lax.dot_general(
        blk, hsa_ref[...], (((1,), (0,)), ((), ())),
        preferred_element_type=jnp.float32)
    deg_r = m_aug[:, d:d + 1] + 1e-8
    w1 = w1_ref[0].astype(jnp.bfloat16)
    hr = jnp.maximum(
        jax.lax.dot_general(
            m_aug[:, :d].astype(jnp.bfloat16), w1, (((1,), (0,)), ((), ())),
            preferred_element_type=jnp.float32),
        0.0) / deg_r
    out_ref[...] = hr

    @pl.when(l < 2)
    def _accumulate():
        hr_aug = jnp.concatenate(
            [hr.astype(jnp.bfloat16), jnp.ones((_C, _PAD), jnp.bfloat16)],
            axis=1)
        tT_ref[...] += jax.lax.dot_general(
            hr_aug, blk, (((0,), (0,)), ((), ())),
            preferred_element_type=jnp.float32)

    @pl.when((l < 2) & (i == nc - 1))
    def _finish_layer():
        @pl.when(l == 0)
        def _save_deg_s():
            degs_ref[...] = tT_ref[d:d + 1, :] + 1e-8
        tT = tT_ref[:d, :] / degs_ref[...]
        w2 = w2_ref[0].astype(jnp.bfloat16)
        hsT = jnp.maximum(
            jax.lax.dot_general(
                w2, tT.astype(jnp.bfloat16), (((0,), (0,)), ((), ())),
                preferred_element_type=jnp.float32),
            0.0)
        hsa_ref[:, :d] = hsT.T.astype(jnp.bfloat16)


@jax.jit
def kernel(h_s, A_rs, r_embed, W_s2r, W_r2s):
    del r_embed
    R, S = A_rs.shape
    d = h_s.shape[1]
    L = W_s2r.shape[0]

    def a_index(l, i):
        return (jnp.where(l == 0, i, 0), 0)

    return pl.pallas_call(
        _gcn_body,
        grid=(L, R // _C),
        in_specs=[
            pl.BlockSpec((_C, S), a_index),
            pl.BlockSpec((S, d), lambda l, i: (0, 0)),
            pl.BlockSpec((1, d, d), lambda l, i: (l, 0, 0)),
            pl.BlockSpec((1, d, d), lambda l, i: (l, 0, 0)),
        ],
        out_specs=pl.BlockSpec((_C, d), lambda l, i: (i, 0)),
        out_shape=jax.ShapeDtypeStruct((R, d), jnp.float32),
        scratch_shapes=[
            pltpu.VMEM((R, S), jnp.bfloat16),
            pltpu.VMEM((S, d + _PAD), jnp.bfloat16),
            pltpu.VMEM((d + _PAD, S), jnp.float32),
            pltpu.VMEM((1, S), jnp.float32),
        ],
    )(A_rs, h_s, W_s2r, W_r2s)


# speedup vs baseline: 1.3295x; 1.0949x over previous
import jax
import jax.numpy as jnp
from jax.experimental import pallas as pl
from jax.experimental.pallas import tpu as pltpu

_C = 2000
_PAD = 8


def _gcn_body(a_ref, hs_ref, w1_ref, w2_ref, out_ref,
              a16_ref, hsa_ref, tT_ref, degs_ref):
    l = pl.program_id(0)
    i = pl.program_id(1)
    nc = pl.num_programs(1)
    S, d = hs_ref.shape

    @pl.when((l == 0) & (i == 0))
    def _init_hs():
        hsa_ref[:, :d] = hs_ref[...].astype(jnp.bfloat16)
        hsa_ref[:, d:] = jnp.ones((S, _PAD), jnp.bfloat16)

    @pl.when(i == 0)
    def _reset_acc():
        tT_ref[...] = jnp.zeros_like(tT_ref)

    @pl.when(l == 0)
    def _stash_block():
        a16_ref[pl.ds(i * _C, _C), :] = a_ref[...].astype(jnp.bfloat16)

    blk = a16_ref[pl.ds(i * _C, _C), :]
    m_aug = jax.lax.dot_general(
        blk, hsa_ref[...], (((1,), (0,)), ((), ())),
        preferred_element_type=jnp.float32)
    deg_r = m_aug[:, d:d + 1] + 1e-8
    w1 = w1_ref[0].astype(jnp.bfloat16)
    hr = jnp.maximum(
        jax.lax.dot_general(
            m_aug[:, :d].astype(jnp.bfloat16), w1, (((1,), (0,)), ((), ())),
            preferred_element_type=jnp.float32),
        0.0) / deg_r

    @pl.when(l == pl.num_programs(0) - 1)
    def _emit():
        out_ref[...] = hr

    @pl.when(l < 2)
    def _accumulate():
        hr_aug = jnp.concatenate(
            [hr.astype(jnp.bfloat16), jnp.ones((_C, _PAD), jnp.bfloat16)],
            axis=1)
        tT_ref[...] += jax.lax.dot_general(
            hr_aug, blk, (((0,), (0,)), ((), ())),
            preferred_element_type=jnp.float32)

    @pl.when((l < 2) & (i == nc - 1))
    def _finish_layer():
        @pl.when(l == 0)
        def _save_deg_s():
            degs_ref[...] = tT_ref[d:d + 1, :] + 1e-8
        tT = tT_ref[:d, :] / degs_ref[...]
        w2 = w2_ref[0].astype(jnp.bfloat16)
        hsT = jnp.maximum(
            jax.lax.dot_general(
                w2, tT.astype(jnp.bfloat16), (((0,), (0,)), ((), ())),
                preferred_element_type=jnp.float32),
            0.0)
        hsa_ref[:, :d] = hsT.T.astype(jnp.bfloat16)


@jax.jit
def kernel(h_s, A_rs, r_embed, W_s2r, W_r2s):
    del r_embed
    R, S = A_rs.shape
    d = h_s.shape[1]
    L = W_s2r.shape[0]

    def a_index(l, i):
        return (jnp.where(l == 0, i, 0), 0)

    return pl.pallas_call(
        _gcn_body,
        grid=(L, R // _C),
        in_specs=[
            pl.BlockSpec((_C, S), a_index),
            pl.BlockSpec((S, d), lambda l, i: (0, 0)),
            pl.BlockSpec((1, d, d), lambda l, i: (l, 0, 0)),
            pl.BlockSpec((1, d, d), lambda l, i: (l, 0, 0)),
        ],
        out_specs=pl.BlockSpec((_C, d),
                               lambda l, i: (jnp.where(l == L - 1, i, 0), 0)),
        out_shape=jax.ShapeDtypeStruct((R, d), jnp.float32),
        scratch_shapes=[
            pltpu.VMEM((R, S), jnp.bfloat16),
            pltpu.VMEM((S, d + _PAD), jnp.bfloat16),
            pltpu.VMEM((d + _PAD, S), jnp.float32),
            pltpu.VMEM((1, S), jnp.float32),
        ],
    )(A_rs, h_s, W_s2r, W_r2s)
